# baseline (speedup 1.0000x reference)
import functools
import math

import jax
import jax.numpy as jnp
from jax.experimental import pallas as pl
from jax.experimental.pallas import tpu as pltpu


def _stats_body(q_ref, mem_ref, wq_ref, bq_ref, wk_ref, bk_ref, wv_ref, bv_ref,
                qout_ref, c_ref, read_ref,
                qs_ref, m_ref, l_ref, acc_ref, *, nb, scale, mb, m_total):
    i = pl.program_id(0)

    @pl.when(i == 0)
    def _init():
        qs_ref[...] = jax.lax.dot_general(
            q_ref[...], wq_ref[...], (((1,), (1,)), ((), ())),
            preferred_element_type=jnp.float32) + bq_ref[...]
        m_ref[...] = jnp.full(m_ref.shape, -jnp.inf, jnp.float32)
        l_ref[...] = jnp.zeros(l_ref.shape, jnp.float32)
        acc_ref[...] = jnp.zeros(acc_ref.shape, jnp.float32)

    row_ok = (jax.lax.broadcasted_iota(jnp.int32, (mb, 1), 0)
              + i * mb) < m_total
    mem = jnp.where(row_ok, mem_ref[...], 0.0)
    k_blk = jax.lax.dot_general(
        mem, wk_ref[...], (((1,), (1,)), ((), ())),
        preferred_element_type=jnp.float32) + bk_ref[...]
    v_blk = jax.lax.dot_general(
        mem, wv_ref[...], (((1,), (1,)), ((), ())),
        preferred_element_type=jnp.float32) + bv_ref[...]
    s = jax.lax.dot_general(
        qs_ref[...], k_blk, (((1,), (1,)), ((), ())),
        preferred_element_type=jnp.float32) * scale
    col_ok = (jax.lax.broadcasted_iota(jnp.int32, (1, mb), 1)
              + i * mb) < m_total
    s = jnp.where(col_ok, s, -jnp.inf)
    m_old = m_ref[...]
    m_new = jnp.maximum(m_old, jnp.max(s, axis=1, keepdims=True))
    p = jnp.exp(s - m_new)
    alpha = jnp.exp(m_old - m_new)
    l_ref[...] = l_ref[...] * alpha + jnp.sum(p, axis=1, keepdims=True)
    acc_ref[...] = acc_ref[...] * alpha + jax.lax.dot_general(
        p, v_blk, (((1,), (0,)), ((), ())), preferred_element_type=jnp.float32)
    m_ref[...] = m_new

    @pl.when(i == nb - 1)
    def _fin():
        qout_ref[...] = qs_ref[...]
        l = l_ref[...]
        c_ref[...] = m_ref[...] + jnp.log(l)
        read_ref[...] = acc_ref[...] / l


def _write_body(q_ref, mem_ref, wk_ref, bk_ref, c_ref, w_ref, *, scale, mb,
                m_total):
    i = pl.program_id(0)
    row_ok = (jax.lax.broadcasted_iota(jnp.int32, (mb, 1), 0)
              + i * mb) < m_total
    mem = jnp.where(row_ok, mem_ref[...], 0.0)
    k_blk = jax.lax.dot_general(
        mem, wk_ref[...], (((1,), (1,)), ((), ())),
        preferred_element_type=jnp.float32) + bk_ref[...]
    s = jax.lax.dot_general(
        q_ref[...], k_blk, (((1,), (1,)), ((), ())),
        preferred_element_type=jnp.float32) * scale
    w_ref[...] = jnp.exp(s - c_ref[...])


def kernel(query, memory, Wq, bq, Wk, bk, Wv, bv):
    B, D = query.shape
    M = memory.shape[0]
    scale = 1.0 / math.sqrt(D)

    mb = 2048
    nb = (M + mb - 1) // mb

    bq2 = bq.reshape(1, D)
    bk2 = bk.reshape(1, D)
    bv2 = bv.reshape(1, D)

    full = lambda shape: pl.BlockSpec(shape, lambda i: (0,) * len(shape))
    f32 = jnp.float32

    q_p, c, read = pl.pallas_call(
        functools.partial(_stats_body, nb=nb, scale=scale, mb=mb, m_total=M),
        grid=(nb,),
        in_specs=[
            full((B, D)),
            pl.BlockSpec((mb, D), lambda i: (i, 0)),
            full((D, D)), full((1, D)),
            full((D, D)), full((1, D)),
            full((D, D)), full((1, D)),
        ],
        out_specs=[full((B, D)), full((B, 1)), full((B, D))],
        out_shape=[
            jax.ShapeDtypeStruct((B, D), f32),
            jax.ShapeDtypeStruct((B, 1), f32),
            jax.ShapeDtypeStruct((B, D), f32),
        ],
        scratch_shapes=[
            pltpu.VMEM((B, D), f32),
            pltpu.VMEM((B, 1), f32),
            pltpu.VMEM((B, 1), f32),
            pltpu.VMEM((B, D), f32),
        ],
        compiler_params=pltpu.CompilerParams(
            dimension_semantics=("arbitrary",)),
    )(query, memory, Wq, bq2, Wk, bk2, Wv, bv2)

    weights = pl.pallas_call(
        functools.partial(_write_body, scale=scale, mb=mb, m_total=M),
        grid=(nb,),
        in_specs=[
            full((B, D)),
            pl.BlockSpec((mb, D), lambda i: (i, 0)),
            full((D, D)), full((1, D)),
            full((B, 1)),
        ],
        out_specs=pl.BlockSpec((B, mb), lambda i: (0, i)),
        out_shape=jax.ShapeDtypeStruct((B, M), f32),
        compiler_params=pltpu.CompilerParams(
            dimension_semantics=("arbitrary",)),
    )(q_p, memory, Wk, bk2, c)

    return (read, weights)
